# baseline (speedup 1.0000x reference)
import functools

import jax
import jax.numpy as jnp
from jax import lax
from jax.experimental import pallas as pl
from jax.experimental.pallas import tpu as pltpu
from jax.experimental.pallas import tpu_sc as plsc

_NC = 2
_NS = 16
_NW = _NC * _NS
_L = 16
_CPT = 16


def _make_sc_kernel(batch, size, hp):
    kpb = batch // _L
    mesh = plsc.VectorSubcoreMesh(core_axis_name="c", subcore_axis_name="s",
                                  num_cores=1)

    @functools.partial(
        pl.kernel,
        out_type=(
            jax.ShapeDtypeStruct((hp, batch), jnp.float32),
            jax.ShapeDtypeStruct((hp, batch), jnp.float32),
        ),
        mesh=mesh,
        scratch_types=[
            pltpu.VMEM((_CPT,), jnp.int32),
            pltpu.VMEM((_CPT, batch), jnp.float32),
            pltpu.VMEM((_CPT, batch), jnp.float32),
            pltpu.VMEM((_CPT, batch), jnp.float32),
            pltpu.VMEM((_CPT, batch), jnp.float32),
            pltpu.SemaphoreType.DMA,
            pltpu.SemaphoreType.DMA,
        ],
    )
    def gather_sigmoid(wt0, wt1, idx, o0, o1, idx_v, g0, g1, ob0, ob1, sem0, sem1):
        wid = lax.axis_index("s") + lax.axis_index("c") * _NS
        j0 = wid * _CPT

        @pl.when(j0 < hp)
        def _active():
            pltpu.sync_copy(idx.at[pl.ds(j0, _CPT)], idx_v)
            pltpu.async_copy(wt0.at[idx_v], g0, sem0)
            pltpu.async_copy(wt1.at[idx_v], g1, sem1)

            def _act(g, ob):
                @pl.loop(0, kpb)
                def _(k):
                    s = pl.ds(k * _L, _L)
                    for c in range(_CPT):
                        ob[c, s] = 1.0 / (1.0 + jnp.exp(-2.0 * g[c, s]))

            pltpu.make_async_copy(wt0.at[idx_v], g0, sem0).wait()
            _act(g0, ob0)
            pltpu.make_async_copy(wt1.at[idx_v], g1, sem1).wait()
            pltpu.sync_copy(ob0, o0.at[pl.ds(j0, _CPT), :])
            _act(g1, ob1)
            pltpu.sync_copy(ob1, o1.at[pl.ds(j0, _CPT), :])

    return gather_sigmoid


def kernel(W0, W1, idx):
    batch, size = W0.shape
    hist = idx.shape[0]
    hp = ((hist + _CPT - 1) // _CPT) * _CPT
    idx32 = idx.astype(jnp.int32)
    if hp != hist:
        idx32 = jnp.concatenate([idx32, jnp.zeros((hp - hist,), jnp.int32)])
    sc = _make_sc_kernel(batch, size, hp)
    o0, o1 = sc(W0.T, W1.T, idx32)
    o0 = o0[:hist].T[..., None]
    o1 = o1[:hist].T[..., None]
    return (o0, o1)
